# baseline (speedup 1.0000x reference)
import functools

import jax
import jax.numpy as jnp
from jax import lax
from jax.experimental import pallas as pl
from jax.experimental.pallas import tpu as pltpu
from jax.experimental.pallas import tpu_sc as plsc

_NC = 2
_NS = 16
_NW = _NC * _NS

_NB = 2
_C = 512


@functools.lru_cache(maxsize=None)
def _make_gather(V, D, N):
    n = N // _NW
    G = n // _C
    M = G // _NB
    mesh = plsc.VectorSubcoreMesh(core_axis_name="c", subcore_axis_name="s")

    @functools.partial(
        pl.kernel,
        mesh=mesh,
        out_type=jax.ShapeDtypeStruct((N, D), jnp.float32),
        scratch_types=[
            pltpu.VMEM((n,), jnp.int32),
            pltpu.VMEM((_NB, _C, D), jnp.float32),
            pltpu.SemaphoreType.DMA,
            pltpu.SemaphoreType.DMA,
            pltpu.SemaphoreType.DMA,
            pltpu.SemaphoreType.DMA,
        ],
        compiler_params=pltpu.CompilerParams(use_tc_tiling_on_sc=False),
    )
    def k(table_hbm, idx_hbm, out_hbm, idx_v, rows_v, gs0, gs1, os0, os1):
        gsem = (gs0, gs1)
        osem = (os0, os1)
        wid = lax.axis_index("s") * _NC + lax.axis_index("c")
        base = wid * n
        pltpu.sync_copy(idx_hbm.at[pl.ds(base, n)], idx_v)

        def fire_gather(cur, b):
            pltpu.async_copy(
                table_hbm.at[idx_v.at[pl.ds(cur * _C, _C)]],
                rows_v.at[b],
                gsem[b],
            )

        def wait_gather(b):
            pltpu.make_async_copy(
                out_hbm.at[pl.ds(0, _C)], rows_v.at[b], gsem[b]
            ).wait()

        def fire_write(cur, b):
            pltpu.async_copy(
                rows_v.at[b],
                out_hbm.at[pl.ds(base + cur * _C, _C)],
                osem[b],
            )

        def wait_write(b):
            pltpu.make_async_copy(
                rows_v.at[b], out_hbm.at[pl.ds(0, _C)], osem[b]
            ).wait()

        for b in range(_NB):
            fire_gather(b, b)

        def body(i, carry):
            for b in range(_NB):
                wait_gather(b)
                fire_write(i * _NB + b, b)
            for b in range(_NB):
                wait_write(b)
                fire_gather((i + 1) * _NB + b, b)
            return carry

        lax.fori_loop(0, M - 1, body, 0)

        for b in range(_NB):
            wait_gather(b)
            fire_write((M - 1) * _NB + b, b)
        for b in range(_NB):
            wait_write(b)

    return k


def kernel(idx_texts, embed_table):
    B, S = idx_texts.shape
    V, D = embed_table.shape
    out = _make_gather(V, D, B * S)(embed_table, idx_texts.reshape(B * S))
    return out.reshape(B, S, D)


# speedup vs baseline: 1.0061x; 1.0061x over previous
import functools

import jax
import jax.numpy as jnp
from jax import lax
from jax.experimental import pallas as pl
from jax.experimental.pallas import tpu as pltpu
from jax.experimental.pallas import tpu_sc as plsc

_NC = 2
_NS = 16
_NW = _NC * _NS

_NB = 4
_GS = 2
_C = 256


@functools.lru_cache(maxsize=None)
def _make_gather(V, D, N):
    n = N // _NW
    G = n // _C
    M = G // _NB
    mesh = plsc.VectorSubcoreMesh(core_axis_name="c", subcore_axis_name="s")

    @functools.partial(
        pl.kernel,
        mesh=mesh,
        out_type=jax.ShapeDtypeStruct((N, D), jnp.float32),
        scratch_types=[
            pltpu.VMEM((n,), jnp.int32),
            pltpu.VMEM((_NB, _C, D), jnp.float32),
            pltpu.SemaphoreType.DMA,
            pltpu.SemaphoreType.DMA,
            pltpu.SemaphoreType.DMA,
            pltpu.SemaphoreType.DMA,
            pltpu.SemaphoreType.DMA,
            pltpu.SemaphoreType.DMA,
            pltpu.SemaphoreType.DMA,
            pltpu.SemaphoreType.DMA,
        ],
        compiler_params=pltpu.CompilerParams(use_tc_tiling_on_sc=False),
    )
    def k(table_hbm, idx_hbm, out_hbm, idx_v, rows_v, *sems):
        gsem = sems[:_NB]
        osem = sems[_NB:]
        wid = lax.axis_index("s") * _NC + lax.axis_index("c")
        base = wid * n
        pltpu.sync_copy(idx_hbm.at[pl.ds(base, n)], idx_v)

        def fire_gather(cur, b):
            pltpu.async_copy(
                table_hbm.at[idx_v.at[pl.ds(cur * _C, _C)]],
                rows_v.at[b],
                gsem[b],
            )

        def wait_gather(b):
            pltpu.make_async_copy(
                out_hbm.at[pl.ds(0, _C)], rows_v.at[b], gsem[b]
            ).wait()

        def fire_write(cur, b):
            pltpu.async_copy(
                rows_v.at[b],
                out_hbm.at[pl.ds(base + cur * _C, _C)],
                osem[b],
            )

        def wait_write(b):
            pltpu.make_async_copy(
                rows_v.at[b], out_hbm.at[pl.ds(0, _C)], osem[b]
            ).wait()

        for b in range(_NB):
            fire_gather(b, b)
            if b >= _GS:
                wait_gather(b - _GS)
                fire_write(b - _GS, b - _GS)

        def body(r, carry):
            for b in range(_NB):
                bj = (b - _GS) % _NB
                i = r * _NB + b
                wait_write(b)
                fire_gather(i, b)
                wait_gather(bj)
                fire_write(i - _GS, bj)
            return carry

        lax.fori_loop(1, M, body, 0)

        for c in range(M * _NB - _GS, M * _NB):
            wait_gather(c % _NB)
            fire_write(c, c % _NB)
        for b in range(_NB):
            wait_write(b)

    return k


def kernel(idx_texts, embed_table):
    B, S = idx_texts.shape
    V, D = embed_table.shape
    out = _make_gather(V, D, B * S)(embed_table, idx_texts.reshape(B * S))
    return out.reshape(B, S, D)


# speedup vs baseline: 1.3392x; 1.3311x over previous
import functools

import jax
import jax.numpy as jnp
from jax import lax
from jax.experimental import pallas as pl
from jax.experimental.pallas import tpu as pltpu
from jax.experimental.pallas import tpu_sc as plsc

_NC = 2
_NS = 16
_NW = _NC * _NS

_NB = 4
_GS = 2
_C = 320


@functools.lru_cache(maxsize=None)
def _make_gather(V, D, N):
    n = N // _NW
    G = n // _C
    M = G // _NB
    mesh = plsc.VectorSubcoreMesh(core_axis_name="c", subcore_axis_name="s")

    @functools.partial(
        pl.kernel,
        mesh=mesh,
        out_type=jax.ShapeDtypeStruct((N, 128), jnp.float32),
        scratch_types=[
            pltpu.VMEM((n,), jnp.int32),
            pltpu.VMEM((_NB, _C, D), jnp.float32),
            pltpu.SemaphoreType.DMA,
            pltpu.SemaphoreType.DMA,
            pltpu.SemaphoreType.DMA,
            pltpu.SemaphoreType.DMA,
            pltpu.SemaphoreType.DMA,
            pltpu.SemaphoreType.DMA,
            pltpu.SemaphoreType.DMA,
            pltpu.SemaphoreType.DMA,
        ],
        compiler_params=pltpu.CompilerParams(use_tc_tiling_on_sc=False),
    )
    def k(table_hbm, idx_hbm, out_hbm, idx_v, rows_v, *sems):
        gsem = sems[:_NB]
        osem = sems[_NB:]
        wid = lax.axis_index("s") * _NC + lax.axis_index("c")
        base = wid * n
        pltpu.sync_copy(idx_hbm.at[pl.ds(base, n)], idx_v)

        def fire_gather(cur, b):
            pltpu.async_copy(
                table_hbm.at[idx_v.at[pl.ds(cur * _C, _C)]],
                rows_v.at[b],
                gsem[b],
            )

        def wait_gather(b):
            pltpu.make_async_copy(
                out_hbm.at[pl.ds(0, _C), pl.ds(0, D)], rows_v.at[b], gsem[b]
            ).wait()

        def fire_write(cur, b):
            pltpu.async_copy(
                rows_v.at[b],
                out_hbm.at[pl.ds(base + cur * _C, _C), pl.ds(0, D)],
                osem[b],
            )

        def wait_write(b):
            pltpu.make_async_copy(
                rows_v.at[b], out_hbm.at[pl.ds(0, _C), pl.ds(0, D)], osem[b]
            ).wait()

        for b in range(_NB):
            fire_gather(b, b)
            if b >= _GS:
                wait_gather(b - _GS)
                fire_write(b - _GS, b - _GS)

        def body(r, carry):
            for b in range(_NB):
                bj = (b - _GS) % _NB
                i = r * _NB + b
                wait_write(b)
                fire_gather(i, b)
                wait_gather(bj)
                fire_write(i - _GS, bj)
            return carry

        lax.fori_loop(1, M, body, 0)

        for c in range(M * _NB - _GS, M * _NB):
            wait_gather(c % _NB)
            fire_write(c, c % _NB)
        for b in range(_NB):
            wait_write(b)

    return k


def kernel(idx_texts, embed_table):
    B, S = idx_texts.shape
    V, D = embed_table.shape
    out = _make_gather(V, D, B * S)(embed_table, idx_texts.reshape(B * S))
    return out.reshape(B, S, 128)[:, :, :D]
